# Initial kernel scaffold; baseline (speedup 1.0000x reference)
import jax
import jax.numpy as jnp
from jax.experimental import pallas as pl


def kernel(X, A, Vs, bs, W1, W2, W3, Wcheb, bcheb, wconv, bconv):
    raise NotImplementedError("write your pallas kernel here")


# speedup vs baseline: 1.7459x; 1.7459x over previous
import jax
import jax.numpy as jnp
from jax.experimental import pallas as pl

N, T1, T2, K = 512, 64, 64, 3


def _stblock_kernel(x_ref, a_ref, vs_ref, bs_ref, w1_ref, w2_ref, w3_ref,
                    wc_ref, bc_ref, wconv_ref, bconv_ref, out_ref):
    x = x_ref[0]
    w1 = w1_ref[0]
    w2 = w2_ref[0]
    w3 = w3_ref[0, 0]

    w1x = jnp.sum(x * w1[None, :], axis=1)
    v = w3 * jnp.sum(x * w2[None, :], axis=1)
    s = w1x[:, None] * v[None, :] + bs_ref[...]

    s = s - jnp.max(s, axis=0, keepdims=True)
    s = jnp.exp(s)
    s = s / jnp.sum(s, axis=0, keepdims=True)

    s = jnp.dot(vs_ref[...], s, preferred_element_type=jnp.float32)

    s = s - jnp.max(s, axis=0, keepdims=True)
    s = jnp.exp(s)
    s = s / jnp.sum(s, axis=0, keepdims=True)

    a_hat = a_ref[...] * s

    row = jax.lax.broadcasted_iota(jnp.int32, (N, N), 0)
    col = jax.lax.broadcasted_iota(jnp.int32, (N, N), 1)
    a_off = jnp.where(row == col, 0.0, a_hat)
    deg = jnp.sum(a_off, axis=1)
    dinv = jnp.where(deg > 0, jax.lax.rsqrt(deg), 0.0)
    l_hat = -(dinv[:, None] * a_off * dinv[None, :])

    lt_dot = lambda m: jax.lax.dot_general(
        l_hat, m, (((0,), (0,)), ((), ())), preferred_element_type=jnp.float32)
    tx1 = lt_dot(x)
    tx2 = 2.0 * lt_dot(tx1) - x

    wc = wc_ref[...]
    out = jnp.dot(x, wc[0], preferred_element_type=jnp.float32)
    out = out + jnp.dot(tx1, wc[1], preferred_element_type=jnp.float32)
    out = out + jnp.dot(tx2, wc[2], preferred_element_type=jnp.float32)
    out = jnp.maximum(out + bc_ref[0][None, :], 0.0)

    wcv = wconv_ref[...]
    t = jax.lax.broadcasted_iota(jnp.int32, (N, T2), 1)
    xl = jnp.where(t >= 1, pltpu_roll(out, 1), 0.0)
    xr = jnp.where(t <= T2 - 2, pltpu_roll(out, -1), 0.0)
    y = wcv[0, 0] * xl + wcv[0, 1] * out + wcv[0, 2] * xr
    y = jnp.maximum(y + bconv_ref[0, 0], 0.0)
    out_ref[0] = y


def pltpu_roll(x, shift):
    return jnp.roll(x, shift, axis=1)


def kernel(X, A, Vs, bs, W1, W2, W3, Wcheb, bcheb, wconv, bconv):
    B = X.shape[0]
    x_hat = X.reshape(B, N, T1)
    w1 = W1.reshape(1, T1)
    w2 = W2.reshape(1, T1)
    w3 = W3.reshape(1, 1)
    bc = bcheb.reshape(1, T2)
    wcv = wconv.reshape(1, K)
    bcv = bconv.reshape(1, 1)

    const = lambda shape: pl.BlockSpec(shape, lambda b: (0,) * len(shape))
    out = pl.pallas_call(
        _stblock_kernel,
        grid=(B,),
        in_specs=[
            pl.BlockSpec((1, N, T1), lambda b: (b, 0, 0)),
            const((N, N)),
            const((N, N)),
            const((N, N)),
            const((1, T1)),
            const((1, T1)),
            const((1, 1)),
            const((K, T1, T2)),
            const((1, T2)),
            const((1, K)),
            const((1, 1)),
        ],
        out_specs=pl.BlockSpec((1, N, T2), lambda b: (b, 0, 0)),
        out_shape=jax.ShapeDtypeStruct((B, N, T2), jnp.float32),
    )(x_hat, A, Vs, bs, w1, w2, w3, Wcheb, bc, wcv, bcv)
    return out.reshape(B, N, 1, T2)
